# Initial kernel scaffold; baseline (speedup 1.0000x reference)
import jax
import jax.numpy as jnp
from jax.experimental import pallas as pl


def kernel(x, table):
    raise NotImplementedError("write your pallas kernel here")


# speedup vs baseline: 6.4929x; 6.4929x over previous
import functools

import jax
import jax.numpy as jnp
from jax import lax
from jax.experimental import pallas as pl
from jax.experimental.pallas import tpu as pltpu
from jax.experimental.pallas import tpu_sc as plsc

NB_CLASSES = 1000
EMB = 50
TS = EMB + 1
V = 20
LANES = 16
NC = 2
NS = 16
NW = NC * NS
NQ = 4


def _sc_pool_kernel(n_rows: int):
    rpw = n_rows // NW
    qrows = rpw // NQ
    qgroups = qrows // LANES
    mesh = plsc.VectorSubcoreMesh(
        core_axis_name="c", subcore_axis_name="s",
        num_cores=NC, num_subcores=NS,
    )

    @functools.partial(
        pl.kernel,
        out_type=jax.ShapeDtypeStruct((EMB * n_rows,), jnp.float32),
        mesh=mesh,
        scratch_types=[
            pltpu.VMEM((NB_CLASSES * TS,), jnp.float32),
            pltpu.VMEM((V * rpw,), jnp.int32),
            pltpu.VMEM((V * LANES,), jnp.float32),
            pltpu.VMEM((2 * EMB * qrows,), jnp.float32),
            pltpu.SemaphoreType.DMA,
        ],
        compiler_params=pltpu.CompilerParams(needs_layout_passes=False),
    )
    def kern(x_hbm, tab_hbm, out_hbm, tab_v, x_v, w_v, out_v, sem):
        wid = lax.axis_index("s") * NC + lax.axis_index("c")
        base = wid * rpw
        pltpu.sync_copy(tab_hbm, tab_v)
        for v in range(V):
            pltpu.sync_copy(x_hbm.at[pl.ds(v * n_rows + base, rpw)],
                            x_v.at[pl.ds(v * rpw, rpw)])

        one = jnp.full((LANES,), 1.0, jnp.float32)

        def qbody(q, _):
            buf = (q % 2) * (EMB * qrows)
            @pl.when(q >= 2)
            def _():
                for j in range(EMB):
                    pltpu.make_async_copy(
                        out_v.at[pl.ds(buf + j * qrows, qrows)],
                        out_hbm.at[pl.ds(j * n_rows + base, qrows)],
                        sem,
                    ).wait()

            def group_body(gl, _):
                r0 = q * qrows + gl * LANES
                ob0 = buf + gl * LANES
                xs = [x_v[pl.ds(v * rpw + r0, LANES)] for v in range(V)]
                cnt = one
                for v in range(1, V):
                    m = xs[v] != xs[0]
                    for u in range(1, v):
                        m = m & (xs[v] != xs[u])
                    w = jnp.where(m, 1.0, 0.0).astype(jnp.float32)
                    w_v[pl.ds(v * LANES, LANES)] = w
                    cnt = cnt + w
                inv = one / cnt

                a0 = xs[0] * TS
                for j in range(EMB):
                    val = plsc.load_gather(tab_v, [a0 + j])
                    out_v[pl.ds(ob0 + j * qrows, LANES)] = inv * val

                def vbody(v, carry):
                    xv = x_v[pl.ds(v * rpw + r0, LANES)]
                    wv = w_v[pl.ds(v * LANES, LANES)] * inv
                    av = xv * TS
                    for j in range(EMB):
                        val = plsc.load_gather(tab_v, [av + j])
                        plsc.addupdate(
                            out_v.at[pl.ds(ob0 + j * qrows, LANES)], wv * val)
                    return carry

                lax.fori_loop(1, V, vbody, 0)
                return 0

            lax.fori_loop(0, qgroups, group_body, 0)

            for j in range(EMB):
                pltpu.make_async_copy(
                    out_v.at[pl.ds(buf + j * qrows, qrows)],
                    out_hbm.at[pl.ds(j * n_rows + base + q * qrows, qrows)],
                    sem,
                ).start()
            return 0

        lax.fori_loop(0, NQ, qbody, 0)
        for _ in range(2 * EMB):
            pltpu.make_async_copy(
                out_v.at[pl.ds(0, qrows)],
                out_hbm.at[pl.ds(base, qrows)],
                sem,
            ).wait()

    return kern


def kernel(x, table):
    b, l, v = x.shape
    n = b * l
    x_t = x.reshape(n, v).astype(jnp.int32).T.reshape(-1)
    tab_p = jnp.pad(table, ((0, 0), (0, 1))).reshape(-1)
    out = _sc_pool_kernel(n)(x_t, tab_p)
    return out.reshape(EMB, n).T.reshape(b, l, EMB)


# speedup vs baseline: 18.8568x; 2.9042x over previous
import functools

import jax
import jax.numpy as jnp
from jax import lax
from jax.experimental import pallas as pl
from jax.experimental.pallas import tpu as pltpu
from jax.experimental.pallas import tpu_sc as plsc

NB_CLASSES = 1000
EMB = 50
TS = EMB + 1
V = 20
LANES = 16
NC = 2
NS = 16
NW = NC * NS
NQ = 4


def _sc_pool_kernel(n_rows: int):
    rpw = n_rows // NW
    qrows = rpw // NQ
    qgroups = qrows // LANES
    mesh = plsc.VectorSubcoreMesh(
        core_axis_name="c", subcore_axis_name="s",
        num_cores=NC, num_subcores=NS,
    )

    @functools.partial(
        pl.kernel,
        out_type=jax.ShapeDtypeStruct((EMB * n_rows,), jnp.float32),
        mesh=mesh,
        scratch_types=[
            pltpu.VMEM((NB_CLASSES * TS,), jnp.float32),
            pltpu.VMEM((V * rpw,), jnp.int32),
            pltpu.VMEM((V * LANES,), jnp.float32),
            pltpu.VMEM((V * LANES,), jnp.int32),
            pltpu.VMEM((2 * EMB * qrows,), jnp.float32),
            pltpu.SemaphoreType.DMA,
        ],
        compiler_params=pltpu.CompilerParams(needs_layout_passes=False),
    )
    def kern(x_hbm, tab_hbm, out_hbm, tab_v, x_v, w_v, a_v, out_v, sem):
        wid = lax.axis_index("s") * NC + lax.axis_index("c")
        base = wid * rpw
        pltpu.sync_copy(tab_hbm, tab_v)
        for v in range(V):
            pltpu.sync_copy(x_hbm.at[pl.ds(v * n_rows + base, rpw)],
                            x_v.at[pl.ds(v * rpw, rpw)])

        one = jnp.full((LANES,), 1.0, jnp.float32)

        def qbody(q, _):
            buf = (q % 2) * (EMB * qrows)
            @pl.when(q >= 2)
            def _():
                for j in range(EMB):
                    pltpu.make_async_copy(
                        out_v.at[pl.ds(buf + j * qrows, qrows)],
                        out_hbm.at[pl.ds(j * n_rows + base, qrows)],
                        sem,
                    ).wait()

            def group_body(gl, _):
                r0 = q * qrows + gl * LANES
                ob0 = buf + gl * LANES
                xs = [x_v[pl.ds(v * rpw + r0, LANES)] for v in range(V)]
                for v in range(V):
                    a_v[pl.ds(v * LANES, LANES)] = xs[v] * TS
                w_v[pl.ds(0, LANES)] = one
                cnt = one
                for v in range(1, V):
                    m = xs[v] != xs[0]
                    for u in range(1, v):
                        m = m & (xs[v] != xs[u])
                    w = jnp.where(m, 1.0, 0.0).astype(jnp.float32)
                    w_v[pl.ds(v * LANES, LANES)] = w
                    cnt = cnt + w
                inv = one / cnt

                for c0 in range(0, EMB, 13):
                    nj = min(13, EMB - c0)
                    accs = [jnp.zeros((LANES,), jnp.float32)] * nj
                    for v in range(V):
                        av = a_v[pl.ds(v * LANES, LANES)]
                        wv = w_v[pl.ds(v * LANES, LANES)] * inv
                        av = av + c0
                        for j in range(nj):
                            val = plsc.load_gather(tab_v, [av + j])
                            accs[j] = accs[j] + wv * val
                    for j in range(nj):
                        out_v[pl.ds(ob0 + (c0 + j) * qrows, LANES)] = accs[j]
                return 0

            lax.fori_loop(0, qgroups, group_body, 0)

            for j in range(EMB):
                pltpu.make_async_copy(
                    out_v.at[pl.ds(buf + j * qrows, qrows)],
                    out_hbm.at[pl.ds(j * n_rows + base + q * qrows, qrows)],
                    sem,
                ).start()
            return 0

        lax.fori_loop(0, NQ, qbody, 0)
        for _ in range(2 * EMB):
            pltpu.make_async_copy(
                out_v.at[pl.ds(0, qrows)],
                out_hbm.at[pl.ds(base, qrows)],
                sem,
            ).wait()

    return kern


def kernel(x, table):
    b, l, v = x.shape
    n = b * l
    x_t = x.reshape(n, v).astype(jnp.int32).T.reshape(-1)
    tab_p = jnp.pad(table, ((0, 0), (0, 1))).reshape(-1)
    out = _sc_pool_kernel(n)(x_t, tab_p)
    return out.reshape(EMB, n).T.reshape(b, l, EMB)
